# baseline (speedup 1.0000x reference)
import functools

import numpy as np

import jax
import jax.numpy as jnp
from jax import lax
from jax.experimental import pallas as pl
from jax.experimental.pallas import tpu as pltpu
from jax.experimental.pallas import tpu_sc as plsc

D = 64
M = 1_000_000
CHUNK = 400
GROUP = 8
NW = 32
NSC = 250
SC_ROWS = NSC * CHUNK
BASE_CH = NSC // NW
EXTRA = NSC - BASE_CH * NW
TBLK = 20000
TLANE = TBLK // 8
TC_OFF = SC_ROWS // TBLK
TC_GRID = (M - SC_ROWS) // TBLK
NEG = -1e30
INV_T = 10.0
THRESH = 0.9


def _sc_partials(q_hbm, qc_hbm, mem_hbm, card_hbm, used_hbm, out_hbm,
                 q_v, qc_v, rows2, card2, used2, part_v, sems):
    w = lax.axis_index("s") * 2 + lax.axis_index("c")
    pltpu.sync_copy(q_hbm, q_v)
    pltpu.sync_copy(qc_hbm, qc_v)
    q0 = q_v[pl.ds(0, 16)]
    q1 = q_v[pl.ds(16, 16)]
    q2 = q_v[pl.ds(32, 16)]
    q3 = q_v[pl.ds(48, 16)]
    qc_vec = qc_v[pl.ds(0, 16)]
    ioti = lax.iota(jnp.int32, 16)
    ohm = [ioti == i for i in range(GROUP)]

    base = w * BASE_CH + jnp.minimum(w, EXTRA)
    n = BASE_CH + jnp.where(w < EXTRA, 1, 0)

    def issue(slot, c):
        r0 = c * CHUNK
        pltpu.async_copy(mem_hbm.at[pl.ds(r0, CHUNK), :], rows2.at[slot],
                         sems.at[slot, 0])
        pltpu.async_copy(card_hbm.at[pl.ds(r0, CHUNK)],
                         card2.at[pl.ds(slot * CHUNK, CHUNK)], sems.at[slot, 1])
        pltpu.async_copy(used_hbm.at[pl.ds(r0, CHUNK)],
                         used2.at[pl.ds(slot * CHUNK, CHUNK)], sems.at[slot, 2])

    def wait(slot, c):
        r0 = c * CHUNK
        pltpu.make_async_copy(mem_hbm.at[pl.ds(r0, CHUNK), :], rows2.at[slot],
                              sems.at[slot, 0]).wait()
        pltpu.make_async_copy(card_hbm.at[pl.ds(r0, CHUNK)],
                              card2.at[pl.ds(slot * CHUNK, CHUNK)],
                              sems.at[slot, 1]).wait()
        pltpu.make_async_copy(used_hbm.at[pl.ds(r0, CHUNK)],
                              used2.at[pl.ds(slot * CHUNK, CHUNK)],
                              sems.at[slot, 2]).wait()

    @pl.when(n > 0)
    def _():
        issue(0, base)

    def chunk_body(i, carry):
        slot = lax.rem(i, 2)
        c = base + i

        @pl.when(i + 1 < n)
        def _():
            issue(1 - slot, c + 1)

        wait(slot, c)

        @plsc.parallel_loop(0, CHUNK // GROUP, carry=carry)
        def group_loop(g, rc):
            m, d_vec, a0, a1, a2, a3 = rc
            gbase = g * GROUP
            cvec = card2[pl.ds(slot * CHUNK + gbase, 16)]
            uvec = used2[pl.ds(slot * CHUNK + gbase, 16)]
            vmaskf = jnp.where(
                (jnp.abs(qc_vec - cvec) <= 1.0) & (uvec > 0.5),
                jnp.float32(1.0), jnp.float32(0.0))
            rows = []
            svals = []
            for i_r in range(GROUP):
                row = gbase + i_r
                r0 = rows2[slot, row, pl.ds(0, 16)]
                r1 = rows2[slot, row, pl.ds(16, 16)]
                r2 = rows2[slot, row, pl.ds(32, 16)]
                r3 = rows2[slot, row, pl.ds(48, 16)]
                rows.append((r0, r1, r2, r3))
                p = (r0 * q0 + r1 * q1) + (r2 * q2 + r3 * q3)
                svals.append(jnp.sum(p))
            sims = jnp.where(ohm[0], svals[0], jnp.float32(0.0))
            for i_r in range(1, GROUP):
                sims = jnp.where(ohm[i_r], svals[i_r], sims)
            validf = jnp.where(sims >= THRESH, vmaskf, jnp.float32(0.0))
            sv = jnp.where(validf > 0.5, sims * INV_T, jnp.float32(NEG))
            new_m = jnp.maximum(m, jnp.max(sv))
            scale = jnp.exp(jnp.full((16,), m - new_m, jnp.float32))
            w_vec = jnp.exp(sv - new_m) * validf
            d_vec = d_vec * scale + w_vec
            a0 = a0 * scale
            a1 = a1 * scale
            a2 = a2 * scale
            a3 = a3 * scale
            for i_r in range(GROUP):
                wi = w_vec[i_r]
                r0, r1, r2, r3 = rows[i_r]
                a0 = a0 + wi * r0
                a1 = a1 + wi * r1
                a2 = a2 + wi * r2
                a3 = a3 + wi * r3
            return (new_m, d_vec, a0, a1, a2, a3)

        return group_loop

    z = jnp.zeros((16,), jnp.float32)
    init = (jnp.float32(NEG), z, z, z, z, z)
    m, d_vec, a0, a1, a2, a3 = lax.fori_loop(0, n, chunk_body, init)

    part_v[pl.ds(0, 16)] = a0
    part_v[pl.ds(16, 16)] = a1
    part_v[pl.ds(32, 16)] = a2
    part_v[pl.ds(48, 16)] = a3
    part_v[pl.ds(64, 16)] = d_vec
    part_v[pl.ds(80, 16)] = jnp.full((16,), m, jnp.float32)
    pltpu.sync_copy(part_v, out_hbm.at[w])


_sc_call = functools.partial(
    pl.kernel,
    out_type=jax.ShapeDtypeStruct((NW, 96), jnp.float32),
    mesh=plsc.VectorSubcoreMesh(core_axis_name="c", subcore_axis_name="s"),
    compiler_params=pltpu.CompilerParams(needs_layout_passes=False),
    scratch_types=[
        pltpu.VMEM((D,), jnp.float32),
        pltpu.VMEM((16,), jnp.float32),
        pltpu.VMEM((2, CHUNK, D), jnp.float32),
        pltpu.VMEM((2 * CHUNK + 16,), jnp.float32),
        pltpu.VMEM((2 * CHUNK + 16,), jnp.float32),
        pltpu.VMEM((96,), jnp.float32),
        pltpu.SemaphoreType.DMA((2, 3)),
    ],
)(_sc_partials)


def _tc_body(q_ref, qc_ref, mem_ref, card_ref, used_ref, out_ref,
             acc_ref, m_ref, d_ref):
    i = pl.program_id(0)

    @pl.when(i == 0)
    def _():
        m_ref[0] = jnp.float32(NEG)
        d_ref[0] = jnp.float32(0.0)
        acc_ref[...] = jnp.zeros((8, D), jnp.float32)

    q8 = q_ref[...]
    mem8 = mem_ref[0]
    sims8 = lax.dot_general(mem8, q8, (((2,), (1,)), ((0,), (0,))))
    qc = qc_ref[0, 0]
    validb = ((sims8 >= THRESH)
              & (jnp.abs(qc - card_ref[0]) <= 1.0)
              & (used_ref[0] > 0.5))
    sv = jnp.where(validb, sims8 * INV_T, jnp.float32(NEG))
    m_old = m_ref[0]
    new_m = jnp.maximum(m_old, jnp.max(sv))
    scale = jnp.exp(m_old - new_m)
    w8 = jnp.where(validb, jnp.exp(sv - new_m), jnp.float32(0.0))
    d_ref[0] = d_ref[0] * scale + jnp.sum(w8)
    contrib = lax.dot_general(w8, mem8, (((1,), (1,)), ((0,), (0,))))
    acc_ref[...] = acc_ref[...] * scale + contrib
    m_ref[0] = new_m

    @pl.when(i == TC_GRID - 1)
    def _():
        acc = jnp.sum(acc_ref[...], axis=0, keepdims=True)
        lane = lax.broadcasted_iota(jnp.int32, (1, 16), 1)
        dvec = jnp.where(lane == 0, d_ref[0], jnp.float32(0.0))
        mvec = jnp.full((1, 16), m_ref[0], jnp.float32)
        out_ref[...] = jnp.concatenate([acc, dvec, mvec], axis=1)


_tc_call = pl.pallas_call(
    _tc_body,
    grid=(TC_GRID,),
    in_specs=[
        pl.BlockSpec((8, D), lambda i: (0, 0)),
        pl.BlockSpec((1, 1), lambda i: (0, 0)),
        pl.BlockSpec((1, 8, TLANE, D), lambda i: (TC_OFF + i, 0, 0, 0)),
        pl.BlockSpec((1, 8, TLANE), lambda i: (i, 0, 0)),
        pl.BlockSpec((1, 8, TLANE), lambda i: (i, 0, 0)),
    ],
    out_specs=pl.BlockSpec((1, 96), lambda i: (0, 0)),
    out_shape=jax.ShapeDtypeStruct((1, 96), jnp.float32),
    scratch_shapes=[
        pltpu.VMEM((8, D), jnp.float32),
        pltpu.SMEM((1,), jnp.float32),
        pltpu.SMEM((1,), jnp.float32),
    ],
)


def _combine_body(p_ref, o_ref):
    p = p_ref[...]
    acc = p[:, :D]
    d_w = jnp.sum(p[:, D:D + 16], axis=1)
    m_w = p[:, D + 16]
    mx = jnp.max(m_w)
    scale = jnp.exp(m_w - mx)
    denom = jnp.sum(d_w * scale)
    num = jnp.sum(acc * scale[:, None], axis=0)
    out = jnp.where(denom > 0, num / jnp.where(denom > 0, denom, 1.0), 0.0)
    o_ref[...] = out.reshape(1, D)


_combine_call = pl.pallas_call(
    _combine_body,
    out_shape=jax.ShapeDtypeStruct((1, D), jnp.float32),
)


@jax.jit
def kernel(query_embedding, query_cardinality, memory_embeddings,
           memory_cardinalities, memory_used):
    q = query_embedding.reshape(D)
    qc = jnp.broadcast_to(query_cardinality.reshape(1), (16,))
    used_f = memory_used.astype(jnp.float32)
    sc_parts = _sc_call(q, qc, memory_embeddings, memory_cardinalities,
                        used_f)
    tc_part = _tc_call(jnp.broadcast_to(query_embedding, (8, D)),
                       query_cardinality.reshape(1, 1),
                       memory_embeddings.reshape(M // TBLK, 8, TLANE, D),
                       memory_cardinalities[SC_ROWS:].reshape(
                           TC_GRID, 8, TLANE),
                       used_f[SC_ROWS:].reshape(TC_GRID, 8, TLANE))
    parts = jnp.concatenate([sc_parts, tc_part], axis=0)
    return _combine_call(parts)


# speedup vs baseline: 3.4877x; 3.4877x over previous
import functools

import numpy as np

import jax
import jax.numpy as jnp
from jax import lax
from jax.experimental import pallas as pl
from jax.experimental.pallas import tpu as pltpu
from jax.experimental.pallas import tpu_sc as plsc

D = 64
M = 1_000_000
CHUNK = 400
GROUP = 8
NW = 32
TBLK = 12800
TC_GRID = 70
TC_ROWS = TBLK * TC_GRID
NSC = (M - TC_ROWS) // CHUNK
SC_CH0 = TC_ROWS // CHUNK
BASE_CH = NSC // NW
EXTRA = NSC - BASE_CH * NW
NEG = -1e30
INV_T = 10.0
THRESH = 0.9


def _sc_partials(q_hbm, qc_hbm, mem_hbm, card_hbm, used_hbm, out_hbm,
                 q_v, qc_v, rows2, card2, used2, part_v, sems):
    w = lax.axis_index("s") * 2 + lax.axis_index("c")
    pltpu.sync_copy(q_hbm, q_v)
    pltpu.sync_copy(qc_hbm, qc_v)
    q0 = q_v[pl.ds(0, 16)]
    q1 = q_v[pl.ds(16, 16)]
    q2 = q_v[pl.ds(32, 16)]
    q3 = q_v[pl.ds(48, 16)]
    qc_vec = qc_v[pl.ds(0, 16)]
    ioti = lax.iota(jnp.int32, 16)
    ohm = [ioti == i for i in range(GROUP)]

    base = SC_CH0 + w * BASE_CH + jnp.minimum(w, EXTRA)
    n = BASE_CH + jnp.where(w < EXTRA, 1, 0)

    def issue(slot, c):
        r0 = c * CHUNK
        pltpu.async_copy(mem_hbm.at[pl.ds(r0, CHUNK), :], rows2.at[slot],
                         sems.at[slot, 0])
        pltpu.async_copy(card_hbm.at[pl.ds(r0, CHUNK)],
                         card2.at[pl.ds(slot * CHUNK, CHUNK)], sems.at[slot, 1])
        pltpu.async_copy(used_hbm.at[pl.ds(r0, CHUNK)],
                         used2.at[pl.ds(slot * CHUNK, CHUNK)], sems.at[slot, 2])

    def wait(slot, c):
        r0 = c * CHUNK
        pltpu.make_async_copy(mem_hbm.at[pl.ds(r0, CHUNK), :], rows2.at[slot],
                              sems.at[slot, 0]).wait()
        pltpu.make_async_copy(card_hbm.at[pl.ds(r0, CHUNK)],
                              card2.at[pl.ds(slot * CHUNK, CHUNK)],
                              sems.at[slot, 1]).wait()
        pltpu.make_async_copy(used_hbm.at[pl.ds(r0, CHUNK)],
                              used2.at[pl.ds(slot * CHUNK, CHUNK)],
                              sems.at[slot, 2]).wait()

    @pl.when(n > 0)
    def _():
        issue(0, base)

    def chunk_body(i, carry):
        slot = lax.rem(i, 2)
        c = base + i

        @pl.when(i + 1 < n)
        def _():
            issue(1 - slot, c + 1)

        wait(slot, c)

        @plsc.parallel_loop(0, CHUNK // GROUP, carry=carry)
        def group_loop(g, rc):
            m, d_vec, a0, a1, a2, a3 = rc
            gbase = g * GROUP
            cvec = card2[pl.ds(slot * CHUNK + gbase, 16)]
            uvec = used2[pl.ds(slot * CHUNK + gbase, 16)]
            vmaskf = jnp.where(
                (jnp.abs(qc_vec - cvec) <= 1.0) & (uvec > 0.5),
                jnp.float32(1.0), jnp.float32(0.0))
            rows = []
            svals = []
            for i_r in range(GROUP):
                row = gbase + i_r
                r0 = rows2[slot, row, pl.ds(0, 16)]
                r1 = rows2[slot, row, pl.ds(16, 16)]
                r2 = rows2[slot, row, pl.ds(32, 16)]
                r3 = rows2[slot, row, pl.ds(48, 16)]
                rows.append((r0, r1, r2, r3))
                p = (r0 * q0 + r1 * q1) + (r2 * q2 + r3 * q3)
                svals.append(jnp.sum(p))
            sims = jnp.where(ohm[0], svals[0], jnp.float32(0.0))
            for i_r in range(1, GROUP):
                sims = jnp.where(ohm[i_r], svals[i_r], sims)
            validf = jnp.where(sims >= THRESH, vmaskf, jnp.float32(0.0))
            sv = jnp.where(validf > 0.5, sims * INV_T, jnp.float32(NEG))
            new_m = jnp.maximum(m, jnp.max(sv))
            scale = jnp.exp(jnp.full((16,), m - new_m, jnp.float32))
            w_vec = jnp.exp(sv - new_m) * validf
            d_vec = d_vec * scale + w_vec
            a0 = a0 * scale
            a1 = a1 * scale
            a2 = a2 * scale
            a3 = a3 * scale
            for i_r in range(GROUP):
                wi = w_vec[i_r]
                r0, r1, r2, r3 = rows[i_r]
                a0 = a0 + wi * r0
                a1 = a1 + wi * r1
                a2 = a2 + wi * r2
                a3 = a3 + wi * r3
            return (new_m, d_vec, a0, a1, a2, a3)

        return group_loop

    z = jnp.zeros((16,), jnp.float32)
    init = (jnp.float32(NEG), z, z, z, z, z)
    m, d_vec, a0, a1, a2, a3 = lax.fori_loop(0, n, chunk_body, init)

    part_v[pl.ds(0, 16)] = a0
    part_v[pl.ds(16, 16)] = a1
    part_v[pl.ds(32, 16)] = a2
    part_v[pl.ds(48, 16)] = a3
    part_v[pl.ds(64, 16)] = d_vec
    part_v[pl.ds(80, 16)] = jnp.full((16,), m, jnp.float32)
    pltpu.sync_copy(part_v, out_hbm.at[w])


_sc_call = functools.partial(
    pl.kernel,
    out_type=jax.ShapeDtypeStruct((NW, 96), jnp.float32),
    mesh=plsc.VectorSubcoreMesh(core_axis_name="c", subcore_axis_name="s"),
    compiler_params=pltpu.CompilerParams(needs_layout_passes=False),
    scratch_types=[
        pltpu.VMEM((D,), jnp.float32),
        pltpu.VMEM((16,), jnp.float32),
        pltpu.VMEM((2, CHUNK, D), jnp.float32),
        pltpu.VMEM((2 * CHUNK + 16,), jnp.float32),
        pltpu.VMEM((2 * CHUNK + 16,), jnp.float32),
        pltpu.VMEM((96,), jnp.float32),
        pltpu.SemaphoreType.DMA((2, 3)),
    ],
)(_sc_partials)


def _tc_body(q_ref, qc_ref, mem_ref, card_ref, used_ref, out_ref,
             acc_ref, m_ref, d_ref):
    i = pl.program_id(0)

    @pl.when(i == 0)
    def _():
        m_ref[...] = jnp.full((1, 1), NEG, jnp.float32)
        d_ref[...] = jnp.zeros((1, 1), jnp.float32)
        acc_ref[...] = jnp.zeros((1, D), jnp.float32)

    qv = q_ref[...]
    mem = mem_ref[...]
    sims = lax.dot_general(qv, mem, (((1,), (1,)), ((), ())))
    s2 = sims.reshape(TBLK // 128, 128)
    qc = qc_ref[...]
    validb = ((s2 >= THRESH)
              & (jnp.abs(qc - card_ref[0]) <= 1.0)
              & (used_ref[0] > 0.5))
    sv = jnp.where(validb, s2 * INV_T, jnp.float32(NEG))
    m_old = m_ref[...]
    new_m = jnp.maximum(m_old, jnp.max(sv, axis=(0, 1), keepdims=True))
    scale = jnp.exp(m_old - new_m)
    w2 = jnp.where(validb, jnp.exp(sv - new_m), jnp.float32(0.0))
    d_ref[...] = d_ref[...] * scale + jnp.sum(w2, axis=(0, 1), keepdims=True)
    wmat = w2.reshape(1, TBLK)
    contrib = lax.dot_general(wmat, mem, (((1,), (0,)), ((), ())))
    acc_ref[...] = acc_ref[...] * scale + contrib
    m_ref[...] = new_m

    @pl.when(i == TC_GRID - 1)
    def _():
        lane = lax.broadcasted_iota(jnp.int32, (1, 16), 1)
        dvec = jnp.where(lane == 0, d_ref[...], jnp.float32(0.0))
        mvec = jnp.broadcast_to(m_ref[...], (1, 16))
        out_ref[...] = jnp.concatenate([acc_ref[...], dvec, mvec], axis=1)


_tc_call = pl.pallas_call(
    _tc_body,
    grid=(TC_GRID,),
    in_specs=[
        pl.BlockSpec((1, D), lambda i: (0, 0)),
        pl.BlockSpec((1, 1), lambda i: (0, 0)),
        pl.BlockSpec((TBLK, D), lambda i: (i, 0)),
        pl.BlockSpec((1, TBLK // 128, 128), lambda i: (i, 0, 0)),
        pl.BlockSpec((1, TBLK // 128, 128), lambda i: (i, 0, 0)),
    ],
    out_specs=pl.BlockSpec((1, 96), lambda i: (0, 0)),
    out_shape=jax.ShapeDtypeStruct((1, 96), jnp.float32),
    scratch_shapes=[
        pltpu.VMEM((1, D), jnp.float32),
        pltpu.VMEM((1, 1), jnp.float32),
        pltpu.VMEM((1, 1), jnp.float32),
    ],
)


def _combine_body(p_ref, o_ref):
    p = p_ref[...]
    acc = p[:, :D]
    d_w = jnp.sum(p[:, D:D + 16], axis=1)
    m_w = p[:, D + 16]
    mx = jnp.max(m_w)
    scale = jnp.exp(m_w - mx)
    denom = jnp.sum(d_w * scale)
    num = jnp.sum(acc * scale[:, None], axis=0)
    out = jnp.where(denom > 0, num / jnp.where(denom > 0, denom, 1.0), 0.0)
    o_ref[...] = out.reshape(1, D)


_combine_call = pl.pallas_call(
    _combine_body,
    out_shape=jax.ShapeDtypeStruct((1, D), jnp.float32),
)


@jax.jit
def kernel(query_embedding, query_cardinality, memory_embeddings,
           memory_cardinalities, memory_used):
    q = query_embedding.reshape(D)
    qc = jnp.broadcast_to(query_cardinality.reshape(1), (16,))
    used_f = memory_used.astype(jnp.float32)
    sc_parts = _sc_call(q, qc, memory_embeddings, memory_cardinalities,
                        used_f)
    tc_part = _tc_call(query_embedding,
                       query_cardinality.reshape(1, 1),
                       memory_embeddings,
                       memory_cardinalities[:TC_ROWS].reshape(
                           TC_GRID, TBLK // 128, 128),
                       used_f[:TC_ROWS].reshape(TC_GRID, TBLK // 128, 128))
    parts = jnp.concatenate([sc_parts, tc_part], axis=0)
    return _combine_call(parts)


# speedup vs baseline: 3.6670x; 1.0514x over previous
import functools

import numpy as np

import jax
import jax.numpy as jnp
from jax import lax
from jax.experimental import pallas as pl
from jax.experimental.pallas import tpu as pltpu
from jax.experimental.pallas import tpu_sc as plsc

D = 64
M = 1_000_000
CHUNK = 400
GROUP = 8
NW = 32
TBLK = 25600
TC_GRID = 33
TC_ROWS = TBLK * TC_GRID
NSC = (M - TC_ROWS) // CHUNK
SC_CH0 = TC_ROWS // CHUNK
BASE_CH = NSC // NW
EXTRA = NSC - BASE_CH * NW
NEG = -1e30
INV_T = 10.0
THRESH = 0.9


def _sc_partials(q_hbm, qc_hbm, mem_hbm, card_hbm, used_hbm, out_hbm,
                 q_v, qc_v, rows2, card2, used2, part_v, sems):
    w = lax.axis_index("s") * 2 + lax.axis_index("c")
    pltpu.sync_copy(q_hbm, q_v)
    pltpu.sync_copy(qc_hbm, qc_v)
    q0 = q_v[pl.ds(0, 16)]
    q1 = q_v[pl.ds(16, 16)]
    q2 = q_v[pl.ds(32, 16)]
    q3 = q_v[pl.ds(48, 16)]
    qc_vec = qc_v[pl.ds(0, 16)]
    ioti = lax.iota(jnp.int32, 16)
    ohm = [ioti == i for i in range(GROUP)]

    base = SC_CH0 + w * BASE_CH + jnp.minimum(w, EXTRA)
    n = BASE_CH + jnp.where(w < EXTRA, 1, 0)

    def issue(slot, c):
        r0 = c * CHUNK
        pltpu.async_copy(mem_hbm.at[pl.ds(r0, CHUNK), :], rows2.at[slot],
                         sems.at[slot, 0])
        pltpu.async_copy(card_hbm.at[pl.ds(r0, CHUNK)],
                         card2.at[pl.ds(slot * CHUNK, CHUNK)], sems.at[slot, 1])
        pltpu.async_copy(used_hbm.at[pl.ds(r0, CHUNK)],
                         used2.at[pl.ds(slot * CHUNK, CHUNK)], sems.at[slot, 2])

    def wait(slot, c):
        r0 = c * CHUNK
        pltpu.make_async_copy(mem_hbm.at[pl.ds(r0, CHUNK), :], rows2.at[slot],
                              sems.at[slot, 0]).wait()
        pltpu.make_async_copy(card_hbm.at[pl.ds(r0, CHUNK)],
                              card2.at[pl.ds(slot * CHUNK, CHUNK)],
                              sems.at[slot, 1]).wait()
        pltpu.make_async_copy(used_hbm.at[pl.ds(r0, CHUNK)],
                              used2.at[pl.ds(slot * CHUNK, CHUNK)],
                              sems.at[slot, 2]).wait()

    @pl.when(n > 0)
    def _():
        issue(0, base)

    def chunk_body(i, carry):
        slot = lax.rem(i, 2)
        c = base + i

        @pl.when(i + 1 < n)
        def _():
            issue(1 - slot, c + 1)

        wait(slot, c)

        @plsc.parallel_loop(0, CHUNK // GROUP, carry=carry)
        def group_loop(g, rc):
            m, d_vec, a0, a1, a2, a3 = rc
            gbase = g * GROUP
            cvec = card2[pl.ds(slot * CHUNK + gbase, 16)]
            uvec = used2[pl.ds(slot * CHUNK + gbase, 16)]
            vmaskf = jnp.where(
                (jnp.abs(qc_vec - cvec) <= 1.0) & (uvec > 0.5),
                jnp.float32(1.0), jnp.float32(0.0))
            rows = []
            svals = []
            for i_r in range(GROUP):
                row = gbase + i_r
                r0 = rows2[slot, row, pl.ds(0, 16)]
                r1 = rows2[slot, row, pl.ds(16, 16)]
                r2 = rows2[slot, row, pl.ds(32, 16)]
                r3 = rows2[slot, row, pl.ds(48, 16)]
                rows.append((r0, r1, r2, r3))
                p = (r0 * q0 + r1 * q1) + (r2 * q2 + r3 * q3)
                svals.append(jnp.sum(p))
            sims = jnp.where(ohm[0], svals[0], jnp.float32(0.0))
            for i_r in range(1, GROUP):
                sims = jnp.where(ohm[i_r], svals[i_r], sims)
            validf = jnp.where(sims >= THRESH, vmaskf, jnp.float32(0.0))
            sv = jnp.where(validf > 0.5, sims * INV_T, jnp.float32(NEG))
            new_m = jnp.maximum(m, jnp.max(sv))
            scale = jnp.exp(jnp.full((16,), m - new_m, jnp.float32))
            w_vec = jnp.exp(sv - new_m) * validf
            d_vec = d_vec * scale + w_vec
            a0 = a0 * scale
            a1 = a1 * scale
            a2 = a2 * scale
            a3 = a3 * scale
            for i_r in range(GROUP):
                wi = w_vec[i_r]
                r0, r1, r2, r3 = rows[i_r]
                a0 = a0 + wi * r0
                a1 = a1 + wi * r1
                a2 = a2 + wi * r2
                a3 = a3 + wi * r3
            return (new_m, d_vec, a0, a1, a2, a3)

        return group_loop

    z = jnp.zeros((16,), jnp.float32)
    init = (jnp.float32(NEG), z, z, z, z, z)
    m, d_vec, a0, a1, a2, a3 = lax.fori_loop(0, n, chunk_body, init)

    part_v[pl.ds(0, 16)] = a0
    part_v[pl.ds(16, 16)] = a1
    part_v[pl.ds(32, 16)] = a2
    part_v[pl.ds(48, 16)] = a3
    part_v[pl.ds(64, 16)] = d_vec
    part_v[pl.ds(80, 16)] = jnp.full((16,), m, jnp.float32)
    pltpu.sync_copy(part_v, out_hbm.at[w])


_sc_call = functools.partial(
    pl.kernel,
    out_type=jax.ShapeDtypeStruct((NW, 96), jnp.float32),
    mesh=plsc.VectorSubcoreMesh(core_axis_name="c", subcore_axis_name="s"),
    compiler_params=pltpu.CompilerParams(needs_layout_passes=False),
    scratch_types=[
        pltpu.VMEM((D,), jnp.float32),
        pltpu.VMEM((16,), jnp.float32),
        pltpu.VMEM((2, CHUNK, D), jnp.float32),
        pltpu.VMEM((2 * CHUNK + 16,), jnp.float32),
        pltpu.VMEM((2 * CHUNK + 16,), jnp.float32),
        pltpu.VMEM((96,), jnp.float32),
        pltpu.SemaphoreType.DMA((2, 3)),
    ],
)(_sc_partials)


def _tc_body(q_ref, qc_ref, mem_ref, card_ref, used_ref, out_ref):
    qv = q_ref[...]
    mem = mem_ref[...]
    sims = lax.dot_general(qv, mem, (((1,), (1,)), ((), ())))
    s2 = sims.reshape(TBLK // 128, 128)
    qc = qc_ref[...]
    validb = ((s2 >= THRESH)
              & (jnp.abs(qc - card_ref[0]) <= 1.0)
              & (used_ref[0] > 0.5))
    sv = jnp.where(validb, s2 * INV_T, jnp.float32(NEG))
    m_b = jnp.max(sv, axis=(0, 1), keepdims=True)
    w2 = jnp.where(validb, jnp.exp(sv - m_b), jnp.float32(0.0))
    d_b = jnp.sum(w2, axis=(0, 1), keepdims=True)
    wmat = w2.reshape(1, TBLK)
    acc_b = lax.dot_general(wmat, mem, (((1,), (0,)), ((), ())))
    lane = lax.broadcasted_iota(jnp.int32, (1, 16), 1)
    dvec = jnp.where(lane == 0, d_b, jnp.float32(0.0))
    mvec = jnp.broadcast_to(m_b, (1, 16))
    out_ref[0] = jnp.concatenate([acc_b, dvec, mvec], axis=1)


_tc_call = pl.pallas_call(
    _tc_body,
    grid=(TC_GRID,),
    in_specs=[
        pl.BlockSpec((1, D), lambda i: (0, 0)),
        pl.BlockSpec((1, 1), lambda i: (0, 0)),
        pl.BlockSpec((TBLK, D), lambda i: (i, 0)),
        pl.BlockSpec((1, TBLK // 128, 128), lambda i: (i, 0, 0)),
        pl.BlockSpec((1, TBLK // 128, 128), lambda i: (i, 0, 0)),
    ],
    out_specs=pl.BlockSpec((1, 1, 96), lambda i: (i, 0, 0)),
    out_shape=jax.ShapeDtypeStruct((TC_GRID, 1, 96), jnp.float32),
)


def _combine_body(p_ref, o_ref):
    p = p_ref[...]
    acc = p[:, :D]
    d_w = jnp.sum(p[:, D:D + 16], axis=1)
    m_w = p[:, D + 16]
    mx = jnp.max(m_w)
    scale = jnp.exp(m_w - mx)
    denom = jnp.sum(d_w * scale)
    num = jnp.sum(acc * scale[:, None], axis=0)
    out = jnp.where(denom > 0, num / jnp.where(denom > 0, denom, 1.0), 0.0)
    o_ref[...] = out.reshape(1, D)


_combine_call = pl.pallas_call(
    _combine_body,
    out_shape=jax.ShapeDtypeStruct((1, D), jnp.float32),
)


@jax.jit
def kernel(query_embedding, query_cardinality, memory_embeddings,
           memory_cardinalities, memory_used):
    q = query_embedding.reshape(D)
    qc = jnp.broadcast_to(query_cardinality.reshape(1), (16,))
    used_f = memory_used.astype(jnp.float32)
    sc_parts = _sc_call(q, qc, memory_embeddings, memory_cardinalities,
                        used_f)
    tc_part = _tc_call(query_embedding,
                       query_cardinality.reshape(1, 1),
                       memory_embeddings,
                       memory_cardinalities[:TC_ROWS].reshape(
                           TC_GRID, TBLK // 128, 128),
                       used_f[:TC_ROWS].reshape(TC_GRID, TBLK // 128, 128))
    parts = jnp.concatenate([sc_parts, tc_part.reshape(TC_GRID, 96)], axis=0)
    return _combine_call(parts)
